# Initial kernel scaffold; baseline (speedup 1.0000x reference)
import jax
import jax.numpy as jnp
from jax.experimental import pallas as pl


def kernel(x, W_enc, b_enc, codebook, W_dec, b_dec):
    raise NotImplementedError("write your pallas kernel here")


# speedup vs baseline: 2.9200x; 2.9200x over previous
import jax
import jax.numpy as jnp
from jax.experimental import pallas as pl

_R = 2048


def _bf(a):
    return a.astype(jnp.bfloat16)


def _vq_kernel(x_ref, w_enc_ref, b_enc_ref, cb_t_ref, w_dec_ref, b_dec_ref, out_ref):
    K = cb_t_ref.shape[1]
    cbt = cb_t_ref[...]
    cbt_b = _bf(cbt).astype(jnp.float32)
    x0 = _bf(x_ref[:, 0:1]).astype(jnp.float32)
    x1 = _bf(x_ref[:, 1:2]).astype(jnp.float32)
    w0 = _bf(w_enc_ref[0:1, :]).astype(jnp.float32)
    w1 = _bf(w_enc_ref[1:2, :]).astype(jnp.float32)
    flat = (x0 * w0 + x1 * w1) + b_enc_ref[...]
    s = jax.lax.dot_general(
        _bf(flat), _bf(cbt), (((1,), (0,)), ((), ())),
        preferred_element_type=jnp.float32,
    )
    t1 = jnp.sum(flat * flat, axis=1, keepdims=True)
    cn = jnp.sum(cbt * cbt, axis=0, keepdims=True)
    d2 = (t1 - 2.0 * s) + cn
    m = jnp.min(d2, axis=1, keepdims=True)
    iota = jax.lax.broadcasted_iota(jnp.int32, d2.shape, 1)
    idx = jnp.min(jnp.where(d2 == m, iota, K), axis=1, keepdims=True)
    wd = _bf(w_dec_ref[...]).astype(jnp.float32)
    dec = jnp.sum(cbt_b * wd, axis=0, keepdims=True)
    sel = jnp.sum(jnp.where(iota == idx, dec, 0.0), axis=1, keepdims=True)
    out_ref[...] = sel + b_dec_ref[0, 0]


def kernel(x, W_enc, b_enc, codebook, W_dec, b_dec):
    T, B, _ = x.shape
    Kc, D = codebook.shape
    n = T * B
    x_flat = x.reshape(n, 2)
    grid = (n // _R,)
    out = pl.pallas_call(
        _vq_kernel,
        grid=grid,
        in_specs=[
            pl.BlockSpec((_R, 2), lambda i: (i, 0)),
            pl.BlockSpec((2, D), lambda i: (0, 0)),
            pl.BlockSpec((1, D), lambda i: (0, 0)),
            pl.BlockSpec((D, Kc), lambda i: (0, 0)),
            pl.BlockSpec((D, 1), lambda i: (0, 0)),
            pl.BlockSpec((1, 1), lambda i: (0, 0)),
        ],
        out_specs=pl.BlockSpec((_R, 1), lambda i: (i, 0)),
        out_shape=jax.ShapeDtypeStruct((n, 1), jnp.float32),
    )(
        x_flat,
        W_enc,
        b_enc.reshape(1, D),
        codebook.T,
        W_dec,
        b_dec.reshape(1, 1),
    )
    return out.reshape(T, B, 1)


# speedup vs baseline: 3.7265x; 1.2762x over previous
import jax
import jax.numpy as jnp
from jax.experimental import pallas as pl

_R = 2048


def _bf(a):
    return a.astype(jnp.bfloat16)


def _vq_kernel(x_ref, w_enc_ref, b_enc_ref, cb_t_ref, w_dec_ref, b_dec_ref, out_ref):
    K = cb_t_ref.shape[1]
    cbt = cb_t_ref[...]
    cbt_b = _bf(cbt).astype(jnp.float32)
    x0 = _bf(x_ref[:, 0:1]).astype(jnp.float32)
    x1 = _bf(x_ref[:, 1:2]).astype(jnp.float32)
    w0 = _bf(w_enc_ref[0:1, :]).astype(jnp.float32)
    w1 = _bf(w_enc_ref[1:2, :]).astype(jnp.float32)
    flat = (x0 * w0 + x1 * w1) + b_enc_ref[...]
    s = jax.lax.dot_general(
        _bf(flat), _bf(cbt), (((1,), (0,)), ((), ())),
        preferred_element_type=jnp.float32,
    )
    cn = jnp.sum(cbt * cbt, axis=0, keepdims=True)
    d2 = cn - 2.0 * s
    m = jnp.min(d2, axis=1, keepdims=True)
    iota = jax.lax.broadcasted_iota(jnp.int32, d2.shape, 1).astype(jnp.float32)
    idx = jnp.min(jnp.where(d2 == m, iota, float(K)), axis=1, keepdims=True)
    wd = _bf(w_dec_ref[...]).astype(jnp.float32)
    dec = jnp.sum(cbt_b * wd, axis=0, keepdims=True)
    sel = jnp.sum(jnp.where(iota == idx, dec, 0.0), axis=1, keepdims=True)
    out_ref[...] = sel + b_dec_ref[0, 0]


def kernel(x, W_enc, b_enc, codebook, W_dec, b_dec):
    T, B, _ = x.shape
    Kc, D = codebook.shape
    n = T * B
    x_flat = x.reshape(n, 2)
    grid = (n // _R,)
    out = pl.pallas_call(
        _vq_kernel,
        grid=grid,
        in_specs=[
            pl.BlockSpec((_R, 2), lambda i: (i, 0)),
            pl.BlockSpec((2, D), lambda i: (0, 0)),
            pl.BlockSpec((1, D), lambda i: (0, 0)),
            pl.BlockSpec((D, Kc), lambda i: (0, 0)),
            pl.BlockSpec((D, 1), lambda i: (0, 0)),
            pl.BlockSpec((1, 1), lambda i: (0, 0)),
        ],
        out_specs=pl.BlockSpec((_R, 1), lambda i: (i, 0)),
        out_shape=jax.ShapeDtypeStruct((n, 1), jnp.float32),
    )(
        x_flat,
        W_enc,
        b_enc.reshape(1, D),
        codebook.T,
        W_dec,
        b_dec.reshape(1, 1),
    )
    return out.reshape(T, B, 1)
